# baseline (speedup 1.0000x reference)
import functools
import jax
import jax.numpy as jnp
from jax import lax
from jax.experimental import pallas as pl
from jax.experimental.pallas import tpu as pltpu
from jax.experimental.pallas import tpu_sc as plsc

EMB = 64
NC, NS, LN = 2, 16, 16
BLK = 2048


def _sc_probe(t4, P):
    mesh = plsc.VectorSubcoreMesh(
        core_axis_name="c", subcore_axis_name="s",
        num_cores=NC, num_subcores=NS)

    n_per_core = P // (NC * BLK)

    @functools.partial(
        pl.kernel,
        out_type=jax.ShapeDtypeStruct((P, 2 * EMB), jnp.float32),
        mesh=mesh,
        scratch_types=[
            pltpu.VMEM_SHARED((BLK, 2 * EMB), jnp.float32),
            pltpu.VMEM_SHARED((BLK, 2 * EMB), jnp.float32),
            pltpu.SemaphoreType.DMA,
            pltpu.SemaphoreType.DMA,
            pltpu.SemaphoreType.DMA,
        ],
    )
    def k(t4_hbm, out_hbm, sp_a, sp_b, isem, osem_a, osem_b):
        cid = lax.axis_index("c")
        sid = lax.axis_index("s")

        @pl.when(sid == 0)
        def _():
            pltpu.async_copy(t4_hbm.at[pl.ds(0, 400)], sp_a.at[pl.ds(0, 400)], isem)
            pltpu.make_async_copy(t4_hbm.at[pl.ds(0, 400)], sp_a.at[pl.ds(0, 400)], isem).wait()
            base_c = cid * n_per_core * BLK

            def o_start(i, sp, osem):
                st = base_c + jnp.minimum(i, n_per_core - 1) * BLK
                pltpu.async_copy(sp, out_hbm.at[pl.ds(st, BLK)], osem)

            def o_wait(i, sp, osem):
                st = base_c + jnp.minimum(i, n_per_core - 1) * BLK
                pltpu.make_async_copy(sp, out_hbm.at[pl.ds(st, BLK)], osem).wait()

            o_start(0, sp_a, osem_a)
            o_start(1, sp_b, osem_b)

            def body(j, _):
                o_wait(2 * j, sp_a, osem_a)
                o_start(2 * j + 2, sp_a, osem_a)
                o_wait(2 * j + 1, sp_b, osem_b)
                o_start(2 * j + 3, sp_b, osem_b)
                return 0

            lax.fori_loop(0, n_per_core // 2 - 1, body, 0)
            o_wait(n_per_core - 2, sp_a, osem_a)
            o_wait(n_per_core - 1, sp_b, osem_b)

    return k(t4)


def kernel(batch, table):
    B, L = batch.shape
    t2 = table[1:L + 1].reshape(L // 2, 2 * EMB)
    t4 = jnp.concatenate([t2, t2, t2, t2], axis=0)
    out = _sc_probe(t4, B * L // 2)
    return out.reshape(B, L, EMB)


# speedup vs baseline: 2.5544x; 2.5544x over previous
import jax
import jax.numpy as jnp
from jax.experimental import pallas as pl
from jax.experimental.pallas import tpu as pltpu

EMB = 64


def _body(b_ref, tflat_ref, out_ref, p_ref):
    L = b_ref.shape[1]
    N = L * EMB

    @pl.when(pl.program_id(0) == 0)
    def _init():
        row = jax.lax.broadcasted_iota(jnp.int32, (L, N), 0)
        col = jax.lax.broadcasted_iota(jnp.int32, (L, N), 1)
        p_ref[...] = (row == col // EMB).astype(jnp.bfloat16)

    mask = (b_ref[...] != 0).astype(jnp.bfloat16)
    y = jax.lax.dot_general(
        mask, p_ref[...],
        dimension_numbers=(((1,), (0,)), ((), ())),
        preferred_element_type=jnp.float32,
    )
    out_ref[...] = y * tflat_ref[...]


def kernel(batch, table):
    B, L = batch.shape
    N = L * EMB
    BB = 256

    tflat = table[1:L + 1].reshape(1, N)

    out = pl.pallas_call(
        _body,
        grid=(B // BB,),
        in_specs=[
            pl.BlockSpec((BB, L), lambda i: (i, 0)),
            pl.BlockSpec((1, N), lambda i: (0, 0)),
        ],
        out_specs=pl.BlockSpec((BB, N), lambda i: (i, 0)),
        out_shape=jax.ShapeDtypeStruct((B, N), jnp.float32),
        scratch_shapes=[pltpu.VMEM((L, N), jnp.bfloat16)],
    )(batch, tflat)
    return out.reshape(B, L, EMB)


# speedup vs baseline: 2.5582x; 1.0015x over previous
import jax
import jax.numpy as jnp
from jax import lax
from jax.experimental import pallas as pl
from jax.experimental.pallas import tpu as pltpu

EMB = 64
KBUF = 4


def _body(b_ref, tflat_ref, out_ref, p_ref, *bufs_and_sems):
    bufs = bufs_and_sems[:KBUF]
    sems = bufs_and_sems[KBUF:]
    L = b_ref.shape[1]
    N = L * EMB
    BB = b_ref.shape[0]
    i = pl.program_id(0)
    ng = pl.num_programs(0)

    @pl.when(i == 0)
    def _init():
        row = lax.broadcasted_iota(jnp.int32, (L, N), 0)
        col = lax.broadcasted_iota(jnp.int32, (L, N), 1)
        p_ref[...] = (row == col // EMB).astype(jnp.bfloat16)

    mask = (b_ref[...] != 0).astype(jnp.bfloat16)
    y = lax.dot_general(
        mask, p_ref[...],
        dimension_numbers=(((1,), (0,)), ((), ())),
        preferred_element_type=jnp.float32,
    ) * tflat_ref[...]

    for s in range(KBUF):
        @pl.when(lax.rem(i, KBUF) == s)
        def _slot(s=s):
            @pl.when(i >= KBUF)
            def _reuse_wait():
                pltpu.make_async_copy(
                    bufs[s], out_ref.at[pl.ds((i - KBUF) * BB, BB)],
                    sems[s]).wait()

            bufs[s][...] = y
            pltpu.make_async_copy(
                bufs[s], out_ref.at[pl.ds(i * BB, BB)], sems[s]).start()

    @pl.when(i == ng - 1)
    def _drain():
        for s in range(KBUF):
            pltpu.make_async_copy(
                bufs[s], out_ref.at[pl.ds(0, BB)], sems[s]).wait()


def kernel(batch, table):
    B, L = batch.shape
    N = L * EMB
    BB = 128

    tflat = table[1:L + 1].reshape(1, N)

    out = pl.pallas_call(
        _body,
        grid=(B // BB,),
        in_specs=[
            pl.BlockSpec((BB, L), lambda i: (i, 0)),
            pl.BlockSpec((1, N), lambda i: (0, 0)),
        ],
        out_specs=pl.BlockSpec(memory_space=pltpu.MemorySpace.HBM),
        out_shape=jax.ShapeDtypeStruct((B, N), jnp.float32),
        scratch_shapes=(
            [pltpu.VMEM((L, N), jnp.bfloat16)]
            + [pltpu.VMEM((BB, N), jnp.float32) for _ in range(KBUF)]
            + [pltpu.SemaphoreType.DMA for _ in range(KBUF)]
        ),
    )(batch, tflat)
    return out.reshape(B, L, EMB)
